# baseline (speedup 1.0000x reference)
import functools

import jax
import jax.numpy as jnp
from jax import lax
from jax.experimental import pallas as pl
from jax.experimental.pallas import tpu as pltpu
from jax.experimental.pallas import tpu_sc as plsc

_LANES = 128
_CH = 4


def _sc_partials(e_map, b, n_seg):
    lanes = _LANES
    njt = b.shape[0]
    r_total = b.shape[1]
    d = njt * 8
    skew = lanes + 1
    info = plsc.get_sparse_core_info()
    nc, ns = info.num_cores, info.num_subcores
    nw = nc * ns
    assert r_total % _CH == 0 and n_seg % 8 == 0
    nch = r_total // _CH
    gb, grem = divmod(nch, nw)
    sg = n_seg // 8
    sb, srem = divmod(sg, ns)
    zr = 8
    while (sb * 8) % zr != 0:
        zr -= 8
    nzchunks = (sb * 8) // zr

    mesh = plsc.VectorSubcoreMesh(core_axis_name="c", subcore_axis_name="s")

    @functools.partial(
        pl.kernel,
        mesh=mesh,
        out_type=(
            jax.ShapeDtypeStruct((nc, n_seg, d), jnp.float32),
            jax.ShapeDtypeStruct((nc, n_seg, d), jnp.float32),
        ),
        scratch_types=[
            pltpu.VMEM_SHARED((n_seg, d), jnp.float32),
            pltpu.VMEM_SHARED((n_seg, d), jnp.float32),
        ] + [pltpu.VMEM((lanes,), jnp.int32) for _ in range(4 * _CH)] + [
            pltpu.VMEM((_CH * lanes, d), jnp.float32),
            pltpu.VMEM((_CH, njt, 8, skew), jnp.float32),
            pltpu.VMEM((_CH, njt, 8, skew), jnp.float32),
            pltpu.VMEM((lanes, d), jnp.float32),
            pltpu.VMEM((zr, d), jnp.float32),
            pltpu.SemaphoreType.DMA,
            pltpu.SemaphoreType.DMA,
            pltpu.SemaphoreType.DMA,
        ],
        compiler_params=pltpu.CompilerParams(
            use_tc_tiling_on_sc=False, needs_layout_passes=False,
            disable_bounds_checks=True),
    )
    def scatter_kernel(e_hbm, b_hbm, sums_out, cnts_out,
                       sums_sh, cnts_sh, *rest):
        idxsets = [rest[k * _CH:(k + 1) * _CH] for k in range(4)]
        idxs0 = idxsets[0]
        rows_v, bt0, bt1, ones_v, zeros_v, lsem0, lsem1, ssem = rest[4 * _CH:]
        cid = lax.axis_index("c")
        sid = lax.axis_index("s")
        wid = cid * ns + sid

        def init_ones(i, carry):
            ones_v[i, :] = jnp.full((d,), 1.0, jnp.float32)
            return carry

        def init_zeros(i, carry):
            zeros_v[i, :] = jnp.zeros((d,), jnp.float32)
            return carry

        lax.fori_loop(0, lanes, init_ones, 0)
        lax.fori_loop(0, zr, init_zeros, 0)

        zlo = (sid * sb + jnp.minimum(sid, srem)) * 8

        def zloop(i, carry):
            o = zlo + i * zr
            pltpu.sync_copy(zeros_v, sums_sh.at[pl.ds(o, zr)])
            pltpu.sync_copy(zeros_v, cnts_sh.at[pl.ds(o, zr)])
            return carry

        lax.fori_loop(0, nzchunks, zloop, 0)

        @pl.when(sid < srem)
        def _():
            o = zlo + sb * 8
            pltpu.sync_copy(zeros_v.at[pl.ds(0, 8)], sums_sh.at[pl.ds(o, 8)])
            pltpu.sync_copy(zeros_v.at[pl.ds(0, 8)], cnts_sh.at[pl.ds(o, 8)])

        plsc.subcore_barrier()

        my_g = gb + jnp.where(wid < grem, 1, 0)
        lo_g = wid * gb + jnp.minimum(wid, grem)

        j16 = jnp.arange(d, dtype=jnp.int32)
        p_jt = j16 // 8
        p_jr = j16 % 8

        def load_copies(ci, idxset, btref, sem):
            g0 = (lo_g + ci) * _CH
            cps = [pltpu.make_async_copy(
                e_hbm.at[pl.ds((g0 + j) * lanes, lanes)], idxset[j], sem)
                for j in range(_CH)]
            for jt in range(njt):
                cps.append(pltpu.make_async_copy(
                    b_hbm.at[jt, pl.ds(g0, _CH)],
                    btref.at[:, jt, :, pl.ds(0, lanes)], sem))
            return cps

        def drain_prev(idxset):
            for j in range(_CH):
                pltpu.make_async_copy(
                    rows_v.at[pl.ds(j * lanes, lanes)],
                    sums_sh.at[idxset[j]], ssem).wait()
                pltpu.make_async_copy(
                    ones_v, cnts_sh.at[idxset[j]], ssem).wait()

        def process(ci, idxset, btref, sem):
            for cp in load_copies(ci, idxset, btref, sem):
                cp.wait()

            @pl.when(ci > 0)
            def _():
                drain_prev(idxset)

            def tloop(g, carry2):
                base = g * lanes
                gvec = jnp.broadcast_to(g, (d,)).astype(jnp.int32)

                @plsc.parallel_loop(0, lanes, step=1, unroll=8)
                def _(c):
                    cvec = jnp.broadcast_to(c, (d,)).astype(jnp.int32)
                    v = plsc.load_gather(btref, [gvec, p_jt, p_jr, cvec])
                    rows_v[base + c, :] = v

                return carry2

            lax.fori_loop(0, _CH, tloop, 0)

            for j in range(_CH):
                pltpu.async_copy(
                    rows_v.at[pl.ds(j * lanes, lanes)],
                    sums_sh.at[idxset[j]], ssem, add=True)
                pltpu.async_copy(ones_v, cnts_sh.at[idxset[j]], ssem,
                                 add=True)

        bts = (bt0, bt1)
        lsems = (lsem0, lsem1)

        @pl.when(my_g > 0)
        def _():
            for cp in load_copies(0, idxsets[0], bts[0], lsems[0]):
                cp.start()

        @pl.when(my_g > 1)
        def _():
            for cp in load_copies(1, idxsets[1], bts[1], lsems[1]):
                cp.start()

        def piter(i4, carry):
            for k in range(4):
                ci = i4 * 4 + k

                @pl.when(ci < my_g)
                def _(ci=ci, k=k):
                    process(ci, idxsets[k], bts[k % 2], lsems[k % 2])

                    @pl.when(ci + 2 < my_g)
                    def _():
                        for cp in load_copies(ci + 2, idxsets[(k + 2) % 4],
                                              bts[k % 2], lsems[k % 2]):
                            cp.start()
            return carry

        lax.fori_loop(0, (my_g + 3) // 4, piter, 0)

        @pl.when(my_g > 0)
        def _():
            drain_prev(idxs0)

        plsc.subcore_barrier()

        pltpu.sync_copy(sums_sh.at[pl.ds(zlo, sb * 8)],
                        sums_out.at[cid, pl.ds(zlo, sb * 8)])
        pltpu.sync_copy(cnts_sh.at[pl.ds(zlo, sb * 8)],
                        cnts_out.at[cid, pl.ds(zlo, sb * 8)])

        @pl.when(sid < srem)
        def _():
            o = zlo + sb * 8
            pltpu.sync_copy(sums_sh.at[pl.ds(o, 8)],
                            sums_out.at[cid, pl.ds(o, 8)])
            pltpu.sync_copy(cnts_sh.at[pl.ds(o, 8)],
                            cnts_out.at[cid, pl.ds(o, 8)])

    return scatter_kernel(e_map, b)


def _tc_combine(sums_p, cnts_p, n_seg, d):
    total = n_seg * d
    lanes = 128
    rows = total // lanes

    def body(s_ref, c_ref, o_ref):
        s = s_ref[0] + s_ref[1]
        c = c_ref[0] + c_ref[1]
        o_ref[...] = s / jnp.maximum(c, 1.0)

    out = pl.pallas_call(
        body,
        out_shape=jax.ShapeDtypeStruct((rows, lanes), jnp.float32),
    )(sums_p.reshape(2, rows, lanes), cnts_p.reshape(2, rows, lanes))
    return out.reshape(n_seg, d)


def kernel(X, ref_a, ref_b, e_map, v_count, Y):
    n_seg = v_count.shape[0]
    d = Y.shape[1]
    groups = e_map.shape[0] // _LANES
    b = Y.T.reshape(d // 8, 8, groups, _LANES).transpose(0, 2, 1, 3)
    sums_p, cnts_p = _sc_partials(e_map, b, n_seg)
    return _tc_combine(sums_p, cnts_p, n_seg, d)


# speedup vs baseline: 1.1419x; 1.1419x over previous
import functools

import jax
import jax.numpy as jnp
from jax import lax
from jax.experimental import pallas as pl
from jax.experimental.pallas import tpu as pltpu
from jax.experimental.pallas import tpu_sc as plsc

_LANES = 128
_CH = 4


def _sc_partials(e_map, b, n_seg):
    lanes = _LANES
    njt = b.shape[0]
    r_total = b.shape[1]
    d = njt * 8
    skew = lanes + 1
    info = plsc.get_sparse_core_info()
    nc, ns = info.num_cores, info.num_subcores
    nw = nc * ns
    assert r_total % _CH == 0 and n_seg % 8 == 0
    nch = r_total // _CH
    gb, grem = divmod(nch, nw)
    sg = n_seg // 8
    sb, srem = divmod(sg, ns)
    zr = 8
    while (sb * 8) % zr != 0:
        zr -= 8
    nzchunks = (sb * 8) // zr

    mesh = plsc.VectorSubcoreMesh(core_axis_name="c", subcore_axis_name="s")

    @functools.partial(
        pl.kernel,
        mesh=mesh,
        out_type=(
            jax.ShapeDtypeStruct((nc, n_seg, d), jnp.float32),
            jax.ShapeDtypeStruct((nc, n_seg, d), jnp.float32),
        ),
        scratch_types=[
            pltpu.VMEM_SHARED((n_seg, d), jnp.float32),
            pltpu.VMEM_SHARED((n_seg, d), jnp.float32),
        ] + [pltpu.VMEM((lanes,), jnp.int32) for _ in range(4 * _CH)] + [
            pltpu.VMEM((_CH * lanes, d), jnp.float32),
            pltpu.VMEM((_CH, njt, 8, skew), jnp.float32),
            pltpu.VMEM((_CH, njt, 8, skew), jnp.float32),
            pltpu.VMEM((lanes, d), jnp.float32),
            pltpu.VMEM((zr, d), jnp.float32),
            pltpu.SemaphoreType.DMA,
            pltpu.SemaphoreType.DMA,
            pltpu.SemaphoreType.DMA,
        ],
        compiler_params=pltpu.CompilerParams(
            use_tc_tiling_on_sc=False, needs_layout_passes=False,
            disable_bounds_checks=True),
    )
    def scatter_kernel(e_hbm, b_hbm, sums_out, cnts_out,
                       sums_sh, cnts_sh, *rest):
        idxsets = [rest[k * _CH:(k + 1) * _CH] for k in range(4)]
        idxs0 = idxsets[0]
        rows_v, bt0, bt1, ones_v, zeros_v, lsem0, lsem1, ssem = rest[4 * _CH:]
        cid = lax.axis_index("c")
        sid = lax.axis_index("s")
        wid = cid * ns + sid

        def init_ones(i, carry):
            ones_v[i, :] = jnp.full((d,), 1.0, jnp.float32)
            return carry

        def init_zeros(i, carry):
            zeros_v[i, :] = jnp.zeros((d,), jnp.float32)
            return carry

        lax.fori_loop(0, lanes, init_ones, 0)
        lax.fori_loop(0, zr, init_zeros, 0)

        zlo = (sid * sb + jnp.minimum(sid, srem)) * 8

        def zloop(i, carry):
            o = zlo + i * zr
            pltpu.sync_copy(zeros_v, sums_sh.at[pl.ds(o, zr)])
            pltpu.sync_copy(zeros_v, cnts_sh.at[pl.ds(o, zr)])
            return carry

        lax.fori_loop(0, nzchunks, zloop, 0)

        @pl.when(sid < srem)
        def _():
            o = zlo + sb * 8
            pltpu.sync_copy(zeros_v.at[pl.ds(0, 8)], sums_sh.at[pl.ds(o, 8)])
            pltpu.sync_copy(zeros_v.at[pl.ds(0, 8)], cnts_sh.at[pl.ds(o, 8)])

        plsc.subcore_barrier()

        my_g = gb + jnp.where(wid < grem, 1, 0)
        lo_g = wid * gb + jnp.minimum(wid, grem)

        j16 = jnp.arange(d, dtype=jnp.int32)
        p_jt = j16 // 8
        p_jr = j16 % 8

        def load_copies(ci, idxset, btref, sem):
            g0 = (lo_g + ci) * _CH
            cps = [pltpu.make_async_copy(
                e_hbm.at[pl.ds((g0 + j) * lanes, lanes)], idxset[j], sem)
                for j in range(_CH)]
            for jt in range(njt):
                cps.append(pltpu.make_async_copy(
                    b_hbm.at[jt, pl.ds(g0, _CH)],
                    btref.at[:, jt, :, pl.ds(0, lanes)], sem))
            return cps

        half = _CH // 2

        def drain_half(idxset, h):
            for j in range(h * half, h * half + half):
                pltpu.make_async_copy(
                    rows_v.at[pl.ds(j * lanes, lanes)],
                    sums_sh.at[idxset[j]], ssem).wait()
                pltpu.make_async_copy(
                    ones_v, cnts_sh.at[idxset[j]], ssem).wait()

        def transpose_half(btref, h):
            def tloop(g, carry2):
                base = g * lanes
                gvec = jnp.broadcast_to(g, (d,)).astype(jnp.int32)

                @plsc.parallel_loop(0, lanes, step=1, unroll=8)
                def _(c):
                    cvec = jnp.broadcast_to(c, (d,)).astype(jnp.int32)
                    v = plsc.load_gather(btref, [gvec, p_jt, p_jr, cvec])
                    rows_v[base + c, :] = v

                return carry2

            lax.fori_loop(h * half, h * half + half, tloop, 0)

        def fire_half(idxset, h):
            for j in range(h * half, h * half + half):
                pltpu.async_copy(
                    rows_v.at[pl.ds(j * lanes, lanes)],
                    sums_sh.at[idxset[j]], ssem, add=True)
                pltpu.async_copy(ones_v, cnts_sh.at[idxset[j]], ssem,
                                 add=True)

        def process(ci, idxset, btref, sem, prev_idxset):
            for cp in load_copies(ci, idxset, btref, sem):
                cp.wait()
            transpose_half(btref, 0)

            @pl.when(ci > 0)
            def _():
                drain_half(prev_idxset, 1)

            fire_half(idxset, 0)
            transpose_half(btref, 1)
            drain_half(idxset, 0)
            fire_half(idxset, 1)

        bts = (bt0, bt1)
        lsems = (lsem0, lsem1)

        @pl.when(my_g > 0)
        def _():
            for cp in load_copies(0, idxsets[0], bts[0], lsems[0]):
                cp.start()

        @pl.when(my_g > 1)
        def _():
            for cp in load_copies(1, idxsets[1], bts[1], lsems[1]):
                cp.start()

        def piter(i4, carry):
            for k in range(4):
                ci = i4 * 4 + k

                @pl.when(ci < my_g)
                def _(ci=ci, k=k):
                    process(ci, idxsets[k], bts[k % 2], lsems[k % 2],
                            idxsets[(k + 3) % 4])

                    @pl.when(ci + 2 < my_g)
                    def _():
                        for cp in load_copies(ci + 2, idxsets[(k + 2) % 4],
                                              bts[k % 2], lsems[k % 2]):
                            cp.start()
            return carry

        lax.fori_loop(0, (my_g + 3) // 4, piter, 0)

        @pl.when(my_g > 0)
        def _():
            drain_half(idxs0, 1)

        plsc.subcore_barrier()

        pltpu.sync_copy(sums_sh.at[pl.ds(zlo, sb * 8)],
                        sums_out.at[cid, pl.ds(zlo, sb * 8)])
        pltpu.sync_copy(cnts_sh.at[pl.ds(zlo, sb * 8)],
                        cnts_out.at[cid, pl.ds(zlo, sb * 8)])

        @pl.when(sid < srem)
        def _():
            o = zlo + sb * 8
            pltpu.sync_copy(sums_sh.at[pl.ds(o, 8)],
                            sums_out.at[cid, pl.ds(o, 8)])
            pltpu.sync_copy(cnts_sh.at[pl.ds(o, 8)],
                            cnts_out.at[cid, pl.ds(o, 8)])

    return scatter_kernel(e_map, b)


def _tc_combine(sums_p, cnts_p, n_seg, d):
    total = n_seg * d
    lanes = 128
    rows = total // lanes

    def body(s_ref, c_ref, o_ref):
        s = s_ref[0] + s_ref[1]
        c = c_ref[0] + c_ref[1]
        o_ref[...] = s / jnp.maximum(c, 1.0)

    out = pl.pallas_call(
        body,
        out_shape=jax.ShapeDtypeStruct((rows, lanes), jnp.float32),
    )(sums_p.reshape(2, rows, lanes), cnts_p.reshape(2, rows, lanes))
    return out.reshape(n_seg, d)


def kernel(X, ref_a, ref_b, e_map, v_count, Y):
    n_seg = v_count.shape[0]
    d = Y.shape[1]
    groups = e_map.shape[0] // _LANES
    b = Y.T.reshape(d // 8, 8, groups, _LANES).transpose(0, 2, 1, 3)
    sums_p, cnts_p = _sc_partials(e_map, b, n_seg)
    return _tc_combine(sums_p, cnts_p, n_seg, d)


# speedup vs baseline: 1.1558x; 1.0122x over previous
import functools

import jax
import jax.numpy as jnp
from jax import lax
from jax.experimental import pallas as pl
from jax.experimental.pallas import tpu as pltpu
from jax.experimental.pallas import tpu_sc as plsc

_LANES = 128
_CH = 4


def _sc_partials(e_map, b, n_seg):
    lanes = _LANES
    njt = b.shape[0]
    r_total = b.shape[1]
    d = njt * 8
    skew = lanes + 1
    info = plsc.get_sparse_core_info()
    nc, ns = info.num_cores, info.num_subcores
    nw = nc * ns
    assert r_total % _CH == 0 and n_seg % 8 == 0
    nch = r_total // _CH
    gb, grem = divmod(nch, nw)
    sg = n_seg // 8
    sb, srem = divmod(sg, ns)
    zr = 8
    while (sb * 8) % zr != 0:
        zr -= 8
    nzchunks = (sb * 8) // zr

    mesh = plsc.VectorSubcoreMesh(core_axis_name="c", subcore_axis_name="s")

    @functools.partial(
        pl.kernel,
        mesh=mesh,
        out_type=(
            jax.ShapeDtypeStruct((nc, n_seg, d), jnp.float32),
            jax.ShapeDtypeStruct((nc, n_seg, d), jnp.float32),
        ),
        scratch_types=[
            pltpu.VMEM_SHARED((n_seg, d), jnp.float32),
            pltpu.VMEM_SHARED((n_seg, d), jnp.float32),
        ] + [pltpu.VMEM((lanes,), jnp.int32) for _ in range(4 * _CH)] + [
            pltpu.VMEM((_CH * lanes, d), jnp.float32),
            pltpu.VMEM((_CH, njt, 8, skew), jnp.float32),
            pltpu.VMEM((_CH, njt, 8, skew), jnp.float32),
            pltpu.VMEM((lanes, d), jnp.float32),
            pltpu.VMEM((zr, d), jnp.float32),
            pltpu.SemaphoreType.DMA,
            pltpu.SemaphoreType.DMA,
            pltpu.SemaphoreType.DMA,
        ],
        compiler_params=pltpu.CompilerParams(
            use_tc_tiling_on_sc=False, needs_layout_passes=False,
            disable_bounds_checks=True),
    )
    def scatter_kernel(e_hbm, b_hbm, sums_out, cnts_out,
                       sums_sh, cnts_sh, *rest):
        idxsets = [rest[k * _CH:(k + 1) * _CH] for k in range(4)]
        idxs0 = idxsets[0]
        rows_v, bt0, bt1, ones_v, zeros_v, lsem0, lsem1, ssem = rest[4 * _CH:]
        cid = lax.axis_index("c")
        sid = lax.axis_index("s")
        wid = cid * ns + sid

        def init_ones(i, carry):
            ones_v[i, :] = jnp.full((d,), 1.0, jnp.float32)
            return carry

        def init_zeros(i, carry):
            zeros_v[i, :] = jnp.zeros((d,), jnp.float32)
            return carry

        lax.fori_loop(0, lanes, init_ones, 0)
        lax.fori_loop(0, zr, init_zeros, 0)

        zlo = (sid * sb + jnp.minimum(sid, srem)) * 8

        def zloop(i, carry):
            o = zlo + i * zr
            pltpu.sync_copy(zeros_v, sums_sh.at[pl.ds(o, zr)])
            pltpu.sync_copy(zeros_v, cnts_sh.at[pl.ds(o, zr)])
            return carry

        lax.fori_loop(0, nzchunks, zloop, 0)

        @pl.when(sid < srem)
        def _():
            o = zlo + sb * 8
            pltpu.sync_copy(zeros_v.at[pl.ds(0, 8)], sums_sh.at[pl.ds(o, 8)])
            pltpu.sync_copy(zeros_v.at[pl.ds(0, 8)], cnts_sh.at[pl.ds(o, 8)])

        plsc.subcore_barrier()

        my_g = gb + jnp.where(wid < grem, 1, 0)
        lo_g = wid * gb + jnp.minimum(wid, grem)

        j16 = jnp.arange(d, dtype=jnp.int32)
        p_jt = j16 // 8
        p_jr = j16 % 8

        def load_copies(ci, idxset, btref, sem):
            g0 = (lo_g + ci) * _CH
            cps = [pltpu.make_async_copy(
                e_hbm.at[pl.ds((g0 + j) * lanes, lanes)], idxset[j], sem)
                for j in range(_CH)]
            for jt in range(njt):
                cps.append(pltpu.make_async_copy(
                    b_hbm.at[jt, pl.ds(g0, _CH)],
                    btref.at[:, jt, :, pl.ds(0, lanes)], sem))
            return cps

        half = _CH // 2

        def drain_half(idxset, h):
            for j in range(h * half, h * half + half):
                pltpu.make_async_copy(
                    rows_v.at[pl.ds(j * lanes, lanes)],
                    sums_sh.at[idxset[j]], ssem).wait()
                pltpu.make_async_copy(
                    ones_v, cnts_sh.at[idxset[j]], ssem).wait()

        def transpose_half(btref, h):
            def tloop(g, carry2):
                base = g * lanes
                gvec = jnp.broadcast_to(g, (d,)).astype(jnp.int32)

                @plsc.parallel_loop(0, lanes, step=1, unroll=16)
                def _(c):
                    cvec = jnp.broadcast_to(c, (d,)).astype(jnp.int32)
                    v = plsc.load_gather(btref, [gvec, p_jt, p_jr, cvec])
                    rows_v[base + c, :] = v

                return carry2

            lax.fori_loop(h * half, h * half + half, tloop, 0)

        def fire_half(idxset, h):
            for j in range(h * half, h * half + half):
                pltpu.async_copy(
                    rows_v.at[pl.ds(j * lanes, lanes)],
                    sums_sh.at[idxset[j]], ssem, add=True)
                pltpu.async_copy(ones_v, cnts_sh.at[idxset[j]], ssem,
                                 add=True)

        def process(ci, idxset, btref, sem, prev_idxset):
            for cp in load_copies(ci, idxset, btref, sem):
                cp.wait()
            transpose_half(btref, 0)

            @pl.when(ci > 0)
            def _():
                drain_half(prev_idxset, 1)

            fire_half(idxset, 0)
            transpose_half(btref, 1)
            drain_half(idxset, 0)
            fire_half(idxset, 1)

        bts = (bt0, bt1)
        lsems = (lsem0, lsem1)

        @pl.when(my_g > 0)
        def _():
            for cp in load_copies(0, idxsets[0], bts[0], lsems[0]):
                cp.start()

        @pl.when(my_g > 1)
        def _():
            for cp in load_copies(1, idxsets[1], bts[1], lsems[1]):
                cp.start()

        def piter(i4, carry):
            for k in range(4):
                ci = i4 * 4 + k

                @pl.when(ci < my_g)
                def _(ci=ci, k=k):
                    process(ci, idxsets[k], bts[k % 2], lsems[k % 2],
                            idxsets[(k + 3) % 4])

                    @pl.when(ci + 2 < my_g)
                    def _():
                        for cp in load_copies(ci + 2, idxsets[(k + 2) % 4],
                                              bts[k % 2], lsems[k % 2]):
                            cp.start()
            return carry

        lax.fori_loop(0, (my_g + 3) // 4, piter, 0)

        @pl.when(my_g > 0)
        def _():
            drain_half(idxs0, 1)

        plsc.subcore_barrier()

        pltpu.sync_copy(sums_sh.at[pl.ds(zlo, sb * 8)],
                        sums_out.at[cid, pl.ds(zlo, sb * 8)])
        pltpu.sync_copy(cnts_sh.at[pl.ds(zlo, sb * 8)],
                        cnts_out.at[cid, pl.ds(zlo, sb * 8)])

        @pl.when(sid < srem)
        def _():
            o = zlo + sb * 8
            pltpu.sync_copy(sums_sh.at[pl.ds(o, 8)],
                            sums_out.at[cid, pl.ds(o, 8)])
            pltpu.sync_copy(cnts_sh.at[pl.ds(o, 8)],
                            cnts_out.at[cid, pl.ds(o, 8)])

    return scatter_kernel(e_map, b)


def _tc_combine(sums_p, cnts_p, n_seg, d):
    total = n_seg * d
    lanes = 128
    rows = total // lanes

    def body(s_ref, c_ref, o_ref):
        s = s_ref[0] + s_ref[1]
        c = c_ref[0] + c_ref[1]
        o_ref[...] = s / jnp.maximum(c, 1.0)

    out = pl.pallas_call(
        body,
        out_shape=jax.ShapeDtypeStruct((rows, lanes), jnp.float32),
    )(sums_p.reshape(2, rows, lanes), cnts_p.reshape(2, rows, lanes))
    return out.reshape(n_seg, d)


def kernel(X, ref_a, ref_b, e_map, v_count, Y):
    n_seg = v_count.shape[0]
    d = Y.shape[1]
    groups = e_map.shape[0] // _LANES
    b = Y.T.reshape(d // 8, 8, groups, _LANES).transpose(0, 2, 1, 3)
    sums_p, cnts_p = _sc_partials(e_map, b, n_seg)
    return _tc_combine(sums_p, cnts_p, n_seg, d)
